# baseline (speedup 1.0000x reference)
import math

import jax
import jax.numpy as jnp
from jax.experimental import pallas as pl
from jax.experimental.pallas import tpu as pltpu

_TEMPERATURE = 0.07
_EPS = 1e-8
_QSCALE = 1.0 / (_TEMPERATURE * math.log(2.0))


def _norm_q_kernel(q_ref, qs_ref):
    q = q_ref[...]
    qn = jnp.sqrt(jnp.sum(q * q, axis=1, keepdims=True))
    qs_ref[...] = (q * (_QSCALE / jnp.maximum(qn, _EPS))).astype(jnp.bfloat16)


def _flash_kernel(qs_ref, a_ref, acc_ref, l_ref):
    j = pl.program_id(0)

    @pl.when(j == 0)
    def _init():
        acc_ref[...] = jnp.zeros_like(acc_ref)
        l_ref[...] = jnp.zeros_like(l_ref)

    a = a_ref[...]
    abf = a.astype(jnp.bfloat16)
    s_raw = jax.lax.dot_general(
        qs_ref[...], abf, (((1,), (1,)), ((), ())),
        preferred_element_type=jnp.float32,
    )
    an = jnp.sqrt(jnp.sum(a * a, axis=1))
    ainv = 1.0 / jnp.maximum(an, _EPS)
    p = jnp.exp2(s_raw * ainv[None, :])
    bn = p.shape[1]
    psum = p[:, 0:128]
    for k in range(1, bn // 128):
        psum = psum + p[:, k * 128:(k + 1) * 128]
    l_ref[...] += psum
    acc_ref[...] += jax.lax.dot_general(
        p.astype(jnp.bfloat16), abf, (((1,), (0,)), ((), ())),
        preferred_element_type=jnp.float32,
    )


def _finalize_kernel(acc_ref, l_ref, o_ref):
    l = jnp.sum(l_ref[...], axis=1, keepdims=True)
    o_ref[...] = acc_ref[...] / l


def kernel(query_address, addresses):
    Q, D = query_address.shape
    N, _ = addresses.shape
    BN = min(2048, N)

    qs = pl.pallas_call(
        _norm_q_kernel,
        out_shape=jax.ShapeDtypeStruct((Q, D), jnp.bfloat16),
    )(query_address)

    acc, l = pl.pallas_call(
        _flash_kernel,
        grid=(N // BN,),
        in_specs=[
            pl.BlockSpec((Q, D), lambda j: (0, 0)),
            pl.BlockSpec((BN, D), lambda j: (j, 0)),
        ],
        out_specs=[
            pl.BlockSpec((Q, D), lambda j: (0, 0)),
            pl.BlockSpec((Q, 128), lambda j: (0, 0)),
        ],
        out_shape=[
            jax.ShapeDtypeStruct((Q, D), jnp.float32),
            jax.ShapeDtypeStruct((Q, 128), jnp.float32),
        ],
        compiler_params=pltpu.CompilerParams(
            dimension_semantics=("arbitrary",),
        ),
    )(qs, addresses)

    return pl.pallas_call(
        _finalize_kernel,
        out_shape=jax.ShapeDtypeStruct((Q, D), jnp.float32),
    )(acc, l)


# speedup vs baseline: 1.0276x; 1.0276x over previous
import math

import jax
import jax.numpy as jnp
from jax.experimental import pallas as pl
from jax.experimental.pallas import tpu as pltpu

_TEMPERATURE = 0.07
_EPS = 1e-8
_QSCALE = 1.0 / (_TEMPERATURE * math.log(2.0))


def _norm_q_kernel(q_ref, qs_ref):
    q = q_ref[...]
    qn = jnp.sqrt(jnp.sum(q * q, axis=1, keepdims=True))
    qs_ref[...] = (q * (_QSCALE / jnp.maximum(qn, _EPS))).astype(jnp.bfloat16)


def _flash_kernel(qs_ref, a_ref, acc_ref, l_ref):
    j = pl.program_id(0)

    @pl.when(j == 0)
    def _init():
        acc_ref[...] = jnp.zeros_like(acc_ref)
        l_ref[...] = jnp.zeros_like(l_ref)

    a = a_ref[...]
    abf = a.astype(jnp.bfloat16)
    s_raw = jax.lax.dot_general(
        qs_ref[...], abf, (((1,), (1,)), ((), ())),
        preferred_element_type=jnp.float32,
    )
    an = jnp.sqrt(jnp.sum(a * a, axis=1))
    ainv = 1.0 / jnp.maximum(an, _EPS)
    p = jnp.exp2(s_raw * ainv[None, :])
    bn = p.shape[1]
    psum = p[:, 0:128]
    for k in range(1, bn // 128):
        psum = psum + p[:, k * 128:(k + 1) * 128]
    l_ref[...] += psum
    acc_ref[...] += jax.lax.dot_general(
        p.astype(jnp.bfloat16), abf, (((1,), (0,)), ((), ())),
        preferred_element_type=jnp.float32,
    )


def _finalize_kernel(acc_ref, l_ref, o_ref):
    l = jnp.sum(l_ref[...], axis=1, keepdims=True)
    o_ref[...] = acc_ref[...] / l


def kernel(query_address, addresses):
    Q, D = query_address.shape
    N, _ = addresses.shape
    BN = min(4096, N)

    qs = pl.pallas_call(
        _norm_q_kernel,
        out_shape=jax.ShapeDtypeStruct((Q, D), jnp.bfloat16),
    )(query_address)

    acc, l = pl.pallas_call(
        _flash_kernel,
        grid=(N // BN,),
        in_specs=[
            pl.BlockSpec((Q, D), lambda j: (0, 0)),
            pl.BlockSpec((BN, D), lambda j: (j, 0)),
        ],
        out_specs=[
            pl.BlockSpec((Q, D), lambda j: (0, 0)),
            pl.BlockSpec((Q, 128), lambda j: (0, 0)),
        ],
        out_shape=[
            jax.ShapeDtypeStruct((Q, D), jnp.float32),
            jax.ShapeDtypeStruct((Q, 128), jnp.float32),
        ],
        compiler_params=pltpu.CompilerParams(
            dimension_semantics=("arbitrary",),
        ),
    )(qs, addresses)

    return pl.pallas_call(
        _finalize_kernel,
        out_shape=jax.ShapeDtypeStruct((Q, D), jnp.float32),
    )(acc, l)


# speedup vs baseline: 1.0371x; 1.0092x over previous
import math

import jax
import jax.numpy as jnp
from jax.experimental import pallas as pl
from jax.experimental.pallas import tpu as pltpu

_TEMPERATURE = 0.07
_EPS = 1e-8
_QSCALE = 1.0 / (_TEMPERATURE * math.log(2.0))


def _norm_q_kernel(q_ref, qs_ref):
    q = q_ref[...]
    qn = jnp.sqrt(jnp.sum(q * q, axis=1, keepdims=True))
    qs_ref[...] = (q * (_QSCALE / jnp.maximum(qn, _EPS))).astype(jnp.bfloat16)


def _flash_kernel(qs_ref, a_ref, acc_ref, l_ref):
    j = pl.program_id(0)

    @pl.when(j == 0)
    def _init():
        acc_ref[...] = jnp.zeros_like(acc_ref)
        l_ref[...] = jnp.zeros_like(l_ref)

    bn_total = a_ref.shape[0]
    for c in range(0, bn_total, 4096):
        a = a_ref[c:c + 4096, :]
        abf = a.astype(jnp.bfloat16)
        s_raw = jax.lax.dot_general(
            qs_ref[...], abf, (((1,), (1,)), ((), ())),
            preferred_element_type=jnp.float32,
        )
        an = jnp.sqrt(jnp.sum(a * a, axis=1))
        ainv = 1.0 / jnp.maximum(an, _EPS)
        p = jnp.exp2(s_raw * ainv[None, :])
        bn = p.shape[1]
        psum = p[:, 0:128]
        for k in range(1, bn // 128):
            psum = psum + p[:, k * 128:(k + 1) * 128]
        l_ref[...] += psum
        acc_ref[...] += jax.lax.dot_general(
            p.astype(jnp.bfloat16), abf, (((1,), (0,)), ((), ())),
            preferred_element_type=jnp.float32,
        )


def _finalize_kernel(acc_ref, l_ref, o_ref):
    l = jnp.sum(l_ref[...], axis=1, keepdims=True)
    o_ref[...] = acc_ref[...] / l


def kernel(query_address, addresses):
    Q, D = query_address.shape
    N, _ = addresses.shape
    BN = min(8192, N)

    qs = pl.pallas_call(
        _norm_q_kernel,
        out_shape=jax.ShapeDtypeStruct((Q, D), jnp.bfloat16),
    )(query_address)

    acc, l = pl.pallas_call(
        _flash_kernel,
        grid=(N // BN,),
        in_specs=[
            pl.BlockSpec((Q, D), lambda j: (0, 0)),
            pl.BlockSpec((BN, D), lambda j: (j, 0)),
        ],
        out_specs=[
            pl.BlockSpec((Q, D), lambda j: (0, 0)),
            pl.BlockSpec((Q, 128), lambda j: (0, 0)),
        ],
        out_shape=[
            jax.ShapeDtypeStruct((Q, D), jnp.float32),
            jax.ShapeDtypeStruct((Q, 128), jnp.float32),
        ],
        compiler_params=pltpu.CompilerParams(
            dimension_semantics=("arbitrary",),
        ),
    )(qs, addresses)

    return pl.pallas_call(
        _finalize_kernel,
        out_shape=jax.ShapeDtypeStruct((Q, D), jnp.float32),
    )(acc, l)


# speedup vs baseline: 1.0506x; 1.0130x over previous
import math

import jax
import jax.numpy as jnp
from jax.experimental import pallas as pl
from jax.experimental.pallas import tpu as pltpu

_TEMPERATURE = 0.07
_EPS = 1e-8
_QSCALE = 1.0 / (_TEMPERATURE * math.log(2.0))


def _norm_q_kernel(q_ref, qs_ref):
    q = q_ref[...]
    qn = jnp.sqrt(jnp.sum(q * q, axis=1, keepdims=True))
    qs_ref[...] = (q * (_QSCALE / jnp.maximum(qn, _EPS))).astype(jnp.bfloat16)


def _flash_kernel(qs_ref, a_ref, o_ref, acc_ref, l_ref):
    j = pl.program_id(0)

    @pl.when(j == 0)
    def _init():
        acc_ref[...] = jnp.zeros_like(acc_ref)
        l_ref[...] = jnp.zeros_like(l_ref)

    bn_total = a_ref.shape[0]
    for c in range(0, bn_total, 4096):
        a = a_ref[c:c + 4096, :]
        abf = a.astype(jnp.bfloat16)
        s_raw = jax.lax.dot_general(
            qs_ref[...], abf, (((1,), (1,)), ((), ())),
            preferred_element_type=jnp.float32,
        )
        an = jnp.sqrt(jnp.sum(a * a, axis=1))
        ainv = 1.0 / jnp.maximum(an, _EPS)
        p = jnp.exp2(s_raw * ainv[None, :])
        bn = p.shape[1]
        psum = p[:, 0:128]
        for k in range(1, bn // 128):
            psum = psum + p[:, k * 128:(k + 1) * 128]
        l_ref[...] += psum
        acc_ref[...] += jax.lax.dot_general(
            p.astype(jnp.bfloat16), abf, (((1,), (0,)), ((), ())),
            preferred_element_type=jnp.float32,
        )

    @pl.when(j == pl.num_programs(0) - 1)
    def _done():
        l = jnp.sum(l_ref[...], axis=1, keepdims=True)
        o_ref[...] = acc_ref[...] / l


def kernel(query_address, addresses):
    Q, D = query_address.shape
    N, _ = addresses.shape
    BN = min(8192, N)

    qs = pl.pallas_call(
        _norm_q_kernel,
        out_shape=jax.ShapeDtypeStruct((Q, D), jnp.bfloat16),
    )(query_address)

    return pl.pallas_call(
        _flash_kernel,
        grid=(N // BN,),
        in_specs=[
            pl.BlockSpec((Q, D), lambda j: (0, 0)),
            pl.BlockSpec((BN, D), lambda j: (j, 0)),
        ],
        out_specs=pl.BlockSpec((Q, D), lambda j: (0, 0)),
        out_shape=jax.ShapeDtypeStruct((Q, D), jnp.float32),
        scratch_shapes=[
            pltpu.VMEM((Q, D), jnp.float32),
            pltpu.VMEM((Q, 128), jnp.float32),
        ],
        compiler_params=pltpu.CompilerParams(
            dimension_semantics=("arbitrary",),
        ),
    )(qs, addresses)


# speedup vs baseline: 1.0530x; 1.0023x over previous
import math

import jax
import jax.numpy as jnp
from jax.experimental import pallas as pl
from jax.experimental.pallas import tpu as pltpu

_TEMPERATURE = 0.07
_EPS = 1e-8
_QSCALE = 1.0 / (_TEMPERATURE * math.log(2.0))


def _norm_q_kernel(q_ref, qs_ref):
    q = q_ref[...]
    qn2 = jnp.sum(q * q, axis=1, keepdims=True)
    qinv = jax.lax.rsqrt(jnp.maximum(qn2, _EPS * _EPS))
    qs_ref[...] = (q * (_QSCALE * qinv)).astype(jnp.bfloat16)


def _flash_kernel(qs_ref, a_ref, o_ref, acc_ref, l_ref):
    j = pl.program_id(0)

    @pl.when(j == 0)
    def _init():
        acc_ref[...] = jnp.zeros_like(acc_ref)
        l_ref[...] = jnp.zeros_like(l_ref)

    bn_total = a_ref.shape[0]
    for c in range(0, bn_total, 2048):
        a = a_ref[c:c + 2048, :]
        abf = a.astype(jnp.bfloat16)
        s_raw = jax.lax.dot_general(
            qs_ref[...], abf, (((1,), (1,)), ((), ())),
            preferred_element_type=jnp.float32,
        )
        nsq = jnp.sum(a * a, axis=1)
        ainv = jax.lax.rsqrt(jnp.maximum(nsq, _EPS * _EPS))
        p = jnp.exp2(s_raw * ainv[None, :])
        bn = p.shape[1]
        psum = p[:, 0:128]
        for k in range(1, bn // 128):
            psum = psum + p[:, k * 128:(k + 1) * 128]
        l_ref[...] += psum
        acc_ref[...] += jax.lax.dot_general(
            p.astype(jnp.bfloat16), abf, (((1,), (0,)), ((), ())),
            preferred_element_type=jnp.float32,
        )

    @pl.when(j == pl.num_programs(0) - 1)
    def _done():
        l = jnp.sum(l_ref[...], axis=1, keepdims=True)
        o_ref[...] = acc_ref[...] / l


def kernel(query_address, addresses):
    Q, D = query_address.shape
    N, _ = addresses.shape
    BN = min(8192, N)

    qs = pl.pallas_call(
        _norm_q_kernel,
        out_shape=jax.ShapeDtypeStruct((Q, D), jnp.bfloat16),
    )(query_address)

    return pl.pallas_call(
        _flash_kernel,
        grid=(N // BN,),
        in_specs=[
            pl.BlockSpec((Q, D), lambda j: (0, 0)),
            pl.BlockSpec((BN, D), lambda j: (j, 0)),
        ],
        out_specs=pl.BlockSpec((Q, D), lambda j: (0, 0)),
        out_shape=jax.ShapeDtypeStruct((Q, D), jnp.float32),
        scratch_shapes=[
            pltpu.VMEM((Q, D), jnp.float32),
            pltpu.VMEM((Q, 128), jnp.float32),
        ],
        compiler_params=pltpu.CompilerParams(
            dimension_semantics=("arbitrary",),
        ),
    )(qs, addresses)
